# baseline (speedup 1.0000x reference)
import jax
import jax.numpy as jnp
from jax.experimental import pallas as pl
from jax.experimental.pallas import tpu as pltpu

B = 1024
D = 8960
E = 6
H = 512
KC = 1792
NK = D // KC
FC = 896
NF = D // FC

_SEGS = ((0, 1024), (1024, 1280), (2304, 3072), (5376, 3072), (8448, 512))

_FILL = []
for _j in range(NF):
    _lo, _hi = _j * FC, (_j + 1) * FC
    _parts = []
    for _i, (_s, _w) in enumerate(_SEGS):
        a, b = max(_lo, _s), min(_hi, _s + _w)
        if a < b:
            _parts.append((_i, a - _s, b - a, a - _lo))
    _FILL.append(tuple(_parts))

_STEP_JS = tuple((2 * _k, 2 * _k + 1) for _k in range(NK))


NSLAB = E * NK
NRING = 2


def _moe_kernel(x0, x1, x2, x3, x4, wg_ref, bg_ref, wexp_hbm, bexp_ref,
                w0_ref, b0_ref, w1_ref, b1_ref, w2_ref, b2_ref,
                wi_ref, bi_ref,
                probs_ref, pred_ref,
                xb_ref, acc_ref, fused_ref, wsel_ref,
                stg0, stg1, wring, wcast_ref, prod_ref, sem0, sem1, wsem):
    e = pl.program_id(0)
    k = pl.program_id(1)
    t = e * NK + k
    bf = jnp.bfloat16
    xrefs = (x0, x1, x2, x3, x4)
    stgs = (stg0, stg1)
    sems = (sem0, sem1)

    def copies(j):
        stg, sem = stgs[j % 2], sems[j % 2]
        return [pltpu.make_async_copy(
                    xrefs[i].at[:, pl.ds(src, w)],
                    stg.at[:, pl.ds(dst, w)], sem)
                for (i, src, w, dst) in _FILL[j]]

    def issue(j):
        for c in copies(j):
            c.start()

    def drain_and_cast(j):
        for c in copies(j):
            c.wait()
        xb_ref[:, pl.ds(j * FC, FC)] = stgs[j % 2][...].astype(bf)

    def w_copy(tt):
        slot = tt % NRING
        et = jnp.minimum(tt // NK, E - 1)
        kt = tt % NK
        return pltpu.make_async_copy(
            wexp_hbm.at[et, pl.ds(pl.multiple_of(kt * KC, KC), KC), :],
            wring.at[slot], wsem.at[slot])

    @pl.when(t == 0)
    def _prologue():
        issue(0)
        issue(1)
        w_copy(0).start()
        w_copy(1).start()
        w_copy(0).wait()
        wcast_ref[0] = wring[0].astype(bf)

    @pl.when(e == 0)
    def _fill():
        for kk, js in enumerate(_STEP_JS):
            @pl.when(k == kk)
            def _(js=js):
                for j in js:
                    drain_and_cast(j)
                    if j + 2 < NF:
                        issue(j + 2)

    ks = pl.multiple_of(k * KC, KC)
    xb_c = xb_ref[:, pl.ds(ks, KC)]
    p_cur = jnp.dot(xb_c, wcast_ref[t % 2],
                    preferred_element_type=jnp.float32)
    prod_ref[t % 2] = p_cur

    @pl.when(k >= 1)
    def _acc():
        prev = jnp.where(k == 1, jnp.zeros_like(acc_ref), acc_ref[...])
        acc_ref[...] = prev + prod_ref[(t - 1) % 2]

    @pl.when(t + 2 < NSLAB)
    def _prefetch_w():
        w_copy(t + 2).start()

    @pl.when(t + 1 < NSLAB)
    def _cast_next():
        w_copy(t + 1).wait()
        wcast_ref[(t + 1) % 2] = wring[(t + 1) % NRING].astype(bf)

    @pl.when(jnp.logical_and(e == 0, k == NK - 1))
    def _gate():
        logits = jnp.dot(xb_ref[...], wg_ref[...].astype(bf),
                         preferred_element_type=jnp.float32) + bg_ref[...]
        m = jnp.max(logits, axis=-1, keepdims=True)
        ex = jnp.exp(logits - m)
        probs = ex / jnp.sum(ex, axis=-1, keepdims=True)
        probs_ref[...] = probs
        iota = jax.lax.broadcasted_iota(jnp.int32, (B, E), 1)
        v1 = jnp.max(probs, axis=-1, keepdims=True)
        i1 = jnp.min(jnp.where(probs == v1, iota, E), axis=-1, keepdims=True)
        masked = jnp.where(iota == i1, -jnp.inf, probs)
        v2 = jnp.max(masked, axis=-1, keepdims=True)
        i2 = jnp.min(jnp.where(masked == v2, iota, E), axis=-1, keepdims=True)
        denom = v1 + v2 + 1e-9
        wsel_ref[...] = (jnp.where(iota == i1, v1 / denom, 0.0)
                         + jnp.where(iota == i2, v2 / denom, 0.0))

    @pl.when(k == NK - 1)
    def _expert_epilogue():
        row_iota = jax.lax.broadcasted_iota(jnp.int32, (E, H), 0)
        b_row = jnp.sum(jnp.where(row_iota == e, bexp_ref[...], 0.0),
                        axis=0, keepdims=True)
        eo = jnp.maximum(acc_ref[...] + p_cur + b_row, 0.0)
        col_iota = jax.lax.broadcasted_iota(jnp.int32, (B, E), 1)
        w_col = jnp.sum(jnp.where(col_iota == e, wsel_ref[...], 0.0),
                        axis=-1, keepdims=True)
        prev = jnp.where(e == 0, jnp.zeros_like(fused_ref), fused_ref[...])
        fused_ref[...] = prev + eo * w_col

    @pl.when(jnp.logical_and(e == E - 1, k == NK - 1))
    def _mlp():
        f = fused_ref[...].astype(bf)
        hid = jnp.tanh(jnp.dot(f, w0_ref[...].astype(bf),
                               preferred_element_type=jnp.float32) + b0_ref[...])
        hid = jnp.tanh(jnp.dot(hid.astype(bf), w1_ref[...].astype(bf),
                               preferred_element_type=jnp.float32) + b1_ref[...])
        hid = jnp.tanh(jnp.dot(hid.astype(bf), w2_ref[...].astype(bf),
                               preferred_element_type=jnp.float32) + b2_ref[...])
        pred_ref[...] = jnp.dot(hid.astype(bf), wi_ref[...].astype(bf),
                                preferred_element_type=jnp.float32) + bi_ref[...]


def kernel(drug_graph, protein_graph, drug_embedding, protein_embedding,
           gene_embedding, W_gate, b_gate, W_exp, b_exp,
           W_out0, b_out0, W_out1, b_out1, W_out2, b_out2, W_int, b_int):
    pinned2 = lambda e, k: (0, 0)
    hbm = pl.BlockSpec(memory_space=pl.ANY)

    probs, pred = pl.pallas_call(
        _moe_kernel,
        grid=(E, NK),
        in_specs=[
            hbm, hbm, hbm, hbm, hbm,
            pl.BlockSpec((D, E), pinned2),
            pl.BlockSpec((1, E), pinned2),
            hbm,
            pl.BlockSpec((E, H), pinned2),
            pl.BlockSpec((512, 1024), pinned2),
            pl.BlockSpec((1, 1024), pinned2),
            pl.BlockSpec((1024, 512), pinned2),
            pl.BlockSpec((1, 512), pinned2),
            pl.BlockSpec((512, 256), pinned2),
            pl.BlockSpec((1, 256), pinned2),
            pl.BlockSpec((256, 2), pinned2),
            pl.BlockSpec((1, 2), pinned2),
        ],
        out_specs=[
            pl.BlockSpec((B, E), pinned2),
            pl.BlockSpec((B, 2), pinned2),
        ],
        out_shape=[
            jax.ShapeDtypeStruct((B, E), jnp.float32),
            jax.ShapeDtypeStruct((B, 2), jnp.float32),
        ],
        scratch_shapes=[
            pltpu.VMEM((B, D), jnp.bfloat16),
            pltpu.VMEM((B, H), jnp.float32),
            pltpu.VMEM((B, H), jnp.float32),
            pltpu.VMEM((B, E), jnp.float32),
            pltpu.VMEM((B, FC), jnp.float32),
            pltpu.VMEM((B, FC), jnp.float32),
            pltpu.VMEM((NRING, KC, H), jnp.float32),
            pltpu.VMEM((2, KC, H), jnp.bfloat16),
            pltpu.VMEM((2, B, H), jnp.float32),
            pltpu.SemaphoreType.DMA,
            pltpu.SemaphoreType.DMA,
            pltpu.SemaphoreType.DMA((NRING,)),
        ],
    )(drug_graph, protein_graph, drug_embedding, protein_embedding,
      gene_embedding, W_gate, b_gate.reshape(1, E), W_exp, b_exp,
      W_out0, b_out0.reshape(1, 1024), W_out1, b_out1.reshape(1, 512),
      W_out2, b_out2.reshape(1, 256), W_int, b_int.reshape(1, 2))
    return (probs, pred)


# speedup vs baseline: 1.0897x; 1.0897x over previous
import jax
import jax.numpy as jnp
from jax.experimental import pallas as pl
from jax.experimental.pallas import tpu as pltpu

B = 1024
D = 8960
E = 6
H = 512
KC = 1792
NK = D // KC
FC = 896
NF = D // FC

_SEGS = ((0, 1024), (1024, 1280), (2304, 3072), (5376, 3072), (8448, 512))

_FILL = []
for _j in range(NF):
    _lo, _hi = _j * FC, (_j + 1) * FC
    _parts = []
    for _i, (_s, _w) in enumerate(_SEGS):
        a, b = max(_lo, _s), min(_hi, _s + _w)
        if a < b:
            _parts.append((_i, a - _s, b - a, a - _lo))
    _FILL.append(tuple(_parts))

_STEP_JS = tuple((2 * _k, 2 * _k + 1) for _k in range(NK))


def _moe_kernel(x0, x1, x2, x3, x4, wg_ref, bg_ref, wexp_ref, bexp_ref,
                w0_ref, b0_ref, w1_ref, b1_ref, w2_ref, b2_ref,
                wi_ref, bi_ref,
                probs_ref, pred_ref,
                xb_ref, acc_ref, fused_ref, wsel_ref,
                stg0, stg1, sem0, sem1):
    e = pl.program_id(0)
    k = pl.program_id(1)
    bf = jnp.bfloat16
    xrefs = (x0, x1, x2, x3, x4)
    stgs = (stg0, stg1)
    sems = (sem0, sem1)

    def copies(j):
        stg, sem = stgs[j % 2], sems[j % 2]
        return [pltpu.make_async_copy(
                    xrefs[i].at[:, pl.ds(src, w)],
                    stg.at[:, pl.ds(dst, w)], sem)
                for (i, src, w, dst) in _FILL[j]]

    def issue(j):
        for c in copies(j):
            c.start()

    def drain_and_cast(j):
        for c in copies(j):
            c.wait()
        xb_ref[:, pl.ds(j * FC, FC)] = stgs[j % 2][...].astype(bf)

    @pl.when(e == 0)
    def _fill():
        for kk, js in enumerate(_STEP_JS):
            @pl.when(k == kk)
            def _(js=js, kk=kk):
                if kk == 0:
                    issue(js[0])
                    issue(js[1])
                for j in js:
                    drain_and_cast(j)
                    if j + 2 < NF:
                        issue(j + 2)

    ks = pl.multiple_of(k * KC, KC)
    xb_c = xb_ref[:, pl.ds(ks, KC)]
    prod = jax.lax.dot_general(xb_c, wexp_ref[0], (((1,), (0,)), ((), ())),
                               preferred_element_type=jnp.float32)
    prev = jnp.where(k == 0, jnp.zeros_like(acc_ref), acc_ref[...])
    acc_ref[...] = prev + prod

    @pl.when(jnp.logical_and(e == 0, k == NK - 1))
    def _gate():
        logits = jnp.dot(xb_ref[...], wg_ref[...].astype(bf),
                         preferred_element_type=jnp.float32) + bg_ref[...]
        m = jnp.max(logits, axis=-1, keepdims=True)
        ex = jnp.exp(logits - m)
        probs = ex / jnp.sum(ex, axis=-1, keepdims=True)
        probs_ref[...] = probs
        iota = jax.lax.broadcasted_iota(jnp.int32, (B, E), 1)
        v1 = jnp.max(probs, axis=-1, keepdims=True)
        i1 = jnp.min(jnp.where(probs == v1, iota, E), axis=-1, keepdims=True)
        masked = jnp.where(iota == i1, -jnp.inf, probs)
        v2 = jnp.max(masked, axis=-1, keepdims=True)
        i2 = jnp.min(jnp.where(masked == v2, iota, E), axis=-1, keepdims=True)
        denom = v1 + v2 + 1e-9
        wsel_ref[...] = (jnp.where(iota == i1, v1 / denom, 0.0)
                         + jnp.where(iota == i2, v2 / denom, 0.0))

    @pl.when(k == NK - 1)
    def _expert_epilogue():
        row_iota = jax.lax.broadcasted_iota(jnp.int32, (E, H), 0)
        b_row = jnp.sum(jnp.where(row_iota == e, bexp_ref[...], 0.0),
                        axis=0, keepdims=True)
        eo = jnp.maximum(acc_ref[...] + b_row, 0.0)
        col_iota = jax.lax.broadcasted_iota(jnp.int32, (B, E), 1)
        w_col = jnp.sum(jnp.where(col_iota == e, wsel_ref[...], 0.0),
                        axis=-1, keepdims=True)
        prev = jnp.where(e == 0, jnp.zeros_like(fused_ref), fused_ref[...])
        fused_ref[...] = prev + eo * w_col

    @pl.when(jnp.logical_and(e == E - 1, k == NK - 1))
    def _mlp():
        f = fused_ref[...].astype(bf)
        hid = jnp.tanh(jnp.dot(f, w0_ref[...].astype(bf),
                               preferred_element_type=jnp.float32) + b0_ref[...])
        hid = jnp.tanh(jnp.dot(hid.astype(bf), w1_ref[...].astype(bf),
                               preferred_element_type=jnp.float32) + b1_ref[...])
        hid = jnp.tanh(jnp.dot(hid.astype(bf), w2_ref[...].astype(bf),
                               preferred_element_type=jnp.float32) + b2_ref[...])
        pred_ref[...] = jnp.dot(hid.astype(bf), wi_ref[...].astype(bf),
                                preferred_element_type=jnp.float32) + bi_ref[...]


def kernel(drug_graph, protein_graph, drug_embedding, protein_embedding,
           gene_embedding, W_gate, b_gate, W_exp, b_exp,
           W_out0, b_out0, W_out1, b_out1, W_out2, b_out2, W_int, b_int):
    pinned2 = lambda e, k: (0, 0)
    hbm = pl.BlockSpec(memory_space=pl.ANY)

    probs, pred = pl.pallas_call(
        _moe_kernel,
        grid=(E, NK),
        in_specs=[
            hbm, hbm, hbm, hbm, hbm,
            pl.BlockSpec((D, E), pinned2),
            pl.BlockSpec((1, E), pinned2),
            pl.BlockSpec((1, KC, H), lambda e, k: (e, k, 0)),
            pl.BlockSpec((E, H), pinned2),
            pl.BlockSpec((512, 1024), pinned2),
            pl.BlockSpec((1, 1024), pinned2),
            pl.BlockSpec((1024, 512), pinned2),
            pl.BlockSpec((1, 512), pinned2),
            pl.BlockSpec((512, 256), pinned2),
            pl.BlockSpec((1, 256), pinned2),
            pl.BlockSpec((256, 2), pinned2),
            pl.BlockSpec((1, 2), pinned2),
        ],
        out_specs=[
            pl.BlockSpec((B, E), pinned2),
            pl.BlockSpec((B, 2), pinned2),
        ],
        out_shape=[
            jax.ShapeDtypeStruct((B, E), jnp.float32),
            jax.ShapeDtypeStruct((B, 2), jnp.float32),
        ],
        scratch_shapes=[
            pltpu.VMEM((B, D), jnp.bfloat16),
            pltpu.VMEM((B, H), jnp.float32),
            pltpu.VMEM((B, H), jnp.float32),
            pltpu.VMEM((B, E), jnp.float32),
            pltpu.VMEM((B, FC), jnp.float32),
            pltpu.VMEM((B, FC), jnp.float32),
            pltpu.SemaphoreType.DMA,
            pltpu.SemaphoreType.DMA,
        ],
    )(drug_graph, protein_graph, drug_embedding, protein_embedding,
      gene_embedding, W_gate, b_gate.reshape(1, E), W_exp, b_exp,
      W_out0, b_out0.reshape(1, 1024), W_out1, b_out1.reshape(1, 512),
      W_out2, b_out2.reshape(1, 256), W_int, b_int.reshape(1, 2))
    return (probs, pred)
